# baseline (speedup 1.0000x reference)
import functools

import jax
import jax.numpy as jnp
from jax import lax
from jax.experimental import pallas as pl
from jax.experimental.pallas import tpu as pltpu
from jax.experimental.pallas import tpu_sc as plsc

BATCH = 1024
HIST = 200
D = 128
B = BATCH * HIST
NC = 2
NS = 16
NW = NC * NS
B_PER_W = B // NW
CHUNK = 64
N_CHUNKS = B_PER_W // CHUNK
NBUF = 10
AHEAD = 6

_mesh = plsc.VectorSubcoreMesh(core_axis_name="c", subcore_axis_name="s")


@functools.partial(
    pl.kernel,
    mesh=_mesh,
    out_type=jax.ShapeDtypeStruct((B, D), jnp.float32),
    scratch_types=(
        [pltpu.VMEM((B_PER_W,), jnp.int32)]
        + [pltpu.VMEM((CHUNK, D), jnp.float32) for _ in range(NBUF)]
        + [pltpu.SemaphoreType.DMA for _ in range(2 * NBUF)]
    ),
)
def _embed(idx_hbm, table_hbm, out_hbm, idx_all, *scr):
    row_bufs = scr[0:NBUF]
    gsems = scr[NBUF:2 * NBUF]
    ssems = scr[2 * NBUF:3 * NBUF]

    wid = lax.axis_index("s") * NC + lax.axis_index("c")
    base = wid * B_PER_W

    pltpu.sync_copy(idx_hbm.at[pl.ds(base, B_PER_W)], idx_all)

    def fire(c, b):
        idx_slice = idx_all.at[pl.ds(c * CHUNK, CHUNK)]
        pltpu.async_copy(table_hbm.at[idx_slice], row_bufs[b], gsems[b])

    def wait_gather(b):
        pltpu.make_async_copy(
            table_hbm.at[idx_all.at[pl.ds(0, CHUNK)]], row_bufs[b],
            gsems[b]).wait()

    def start_store(c, b):
        off = base + c * CHUNK
        pltpu.async_copy(row_bufs[b], out_hbm.at[pl.ds(off, CHUNK)], ssems[b])

    def wait_store(b):
        pltpu.make_async_copy(
            row_bufs[b], out_hbm.at[pl.ds(base, CHUNK)], ssems[b]).wait()

    for c in range(AHEAD):
        fire(c, c)

    @pl.loop(0, N_CHUNKS, step=NBUF)
    def _(g):
        for b in range(NBUF):
            c = g + b
            pf = c + AHEAD
            bpf = (b + AHEAD) % NBUF

            @pl.when(pf >= NBUF)
            def _():
                wait_store(bpf)

            @pl.when(pf < N_CHUNKS)
            def _():
                fire(pf, bpf)

            wait_gather(b)
            start_store(c, b)

    for c in range(N_CHUNKS - (NBUF - AHEAD), N_CHUNKS):
        wait_store(c % NBUF)


def kernel(word_ids, table):
    idx = word_ids.reshape(B).astype(jnp.int32)
    out = _embed(idx, table)
    return out.reshape(BATCH, HIST, D)
